# baseline (speedup 1.0000x reference)
import functools

import jax
import jax.numpy as jnp
from jax import lax
from jax.experimental import pallas as pl
from jax.experimental.pallas import tpu as pltpu
from jax.experimental.pallas import tpu_sc as plsc

_NC = 2
_NS = 16
_CH = 128
_RCH = 128


def _npad(n):
  return (n // (_NS * _RCH) + 1) * (_NS * _RCH)


def _pad_edges(edge_index, n):
  e = edge_index.shape[1]
  nw = _NC * _NS
  nchunks = -(-e // _CH)
  cpt8 = -(-(-(-nchunks // nw)) // 8) * 8
  epad = nw * cpt8 * _CH
  if epad > e:
    npad = _npad(n)
    idx = jnp.arange(epad - e, dtype=jnp.int32)
    dsts = n + idx % (npad - n)
    srcs = idx % n
    pad = jnp.stack([srcs, dsts], axis=0)
    edge_index = jnp.concatenate([edge_index, pad], axis=1)
  return edge_index, cpt8


def _sc_segment_sum(x, ei_r, cpt8):
  n, d = x.shape
  npad = _npad(n)
  rpt = npad // _NS
  nr = rpt // _RCH
  assert d % 16 == 0, d

  mesh = plsc.VectorSubcoreMesh(core_axis_name="c", subcore_axis_name="s")

  @functools.partial(
      pl.kernel,
      out_type=jax.ShapeDtypeStruct((_NC, npad, d), jnp.float32),
      mesh=mesh,
      scratch_types=[
          [pltpu.VMEM((_CH,), jnp.int32)] * 2,
          pltpu.VMEM((_CH,), jnp.int32),
          pltpu.VMEM((_CH, d), jnp.float32),
          pltpu.VMEM((_RCH, d), jnp.float32),
          pltpu.VMEM_SHARED((npad, d), jnp.float32),
          [pltpu.SemaphoreType.DMA] * 2,
          pltpu.SemaphoreType.DMA,
          pltpu.SemaphoreType.DMA,
      ],
  )
  def k(x_hbm, ei_hbm, out_hbm, sidx, didx, rows, zb, acc, semi, semd, semg):
    cid = lax.axis_index("c")
    sid = lax.axis_index("s")
    wid = sid * _NC + cid
    c0 = wid * cpt8

    nw = _NC * _NS

    def cb(t):
      return (wid + nw * t) * _CH

    def sidx_start(t, b):
      pltpu.async_copy(ei_hbm.at[0, pl.ds(cb(t), _CH)], sidx[b], semi[b])

    def sidx_wait(t, b):
      pltpu.make_async_copy(ei_hbm.at[0, pl.ds(cb(t), _CH)], sidx[b],
                            semi[b]).wait()

    def didx_start(t):
      pltpu.async_copy(ei_hbm.at[1, pl.ds(cb(t), _CH)], didx, semd)

    def didx_wait(t):
      pltpu.make_async_copy(ei_hbm.at[1, pl.ds(cb(t), _CH)], didx,
                            semd).wait()


    def zrow(i, carry):
      for j in range(d // 16):
        zb[i, pl.ds(j * 16, 16)] = jnp.zeros((16,), jnp.float32)
      return carry
    lax.fori_loop(0, _RCH, zrow, 0)
    r0 = sid * rpt
    for kk in range(nr):
      pltpu.sync_copy(zb, acc.at[pl.ds(r0 + kk * _RCH, _RCH)])
    plsc.subcore_barrier()

    @plsc.parallel_loop(0, cpt8, step=1, unroll=2, carry=jnp.int32(0))
    def _loop(t, carry):
      pltpu.sync_copy(ei_hbm.at[0, pl.ds(cb(t), _CH)], sidx[0])
      pltpu.sync_copy(ei_hbm.at[1, pl.ds(cb(t), _CH)], didx)
      pltpu.async_copy(x_hbm.at[sidx[0]], rows, semg).wait()
      pltpu.sync_copy(rows, acc.at[didx], add=True)
      return carry
    plsc.subcore_barrier()

    for kk in range(nr):
      pltpu.sync_copy(acc.at[pl.ds(r0 + kk * _RCH, _RCH)],
                      out_hbm.at[cid, pl.ds(r0 + kk * _RCH, _RCH)])

  return k(x, ei_r)


def _matmul_t(h, w):
  return lax.dot_general(h, w, (((1,), (1,)), ((), ())),
                         preferred_element_type=jnp.float32)


def _gin_mlp(h, w1, b1, g, be, w2, b2):
  h = _matmul_t(h, w1) + b1
  m = jnp.mean(h, axis=0, keepdims=True)
  v = jnp.mean((h - m) * (h - m), axis=0, keepdims=True)
  h = (h - m) * lax.rsqrt(v + 1e-5) * g + be
  h = jnp.maximum(h, 0.0)
  h = _matmul_t(h, w2) + b2
  return jnp.maximum(h, 0.0)


def _tc_layer_a(x, p, w1, b1, g, be, w2, b2):
  n, d = x.shape

  def body(x_ref, p_ref, w1_ref, b1_ref, g_ref, be_ref, w2_ref, b2_ref, o_ref):
    h = x_ref[...] + p_ref[0, :n, :] + p_ref[1, :n, :]
    o_ref[...] = _gin_mlp(h, w1_ref[...], b1_ref[...], g_ref[...],
                          be_ref[...], w2_ref[...], b2_ref[...])

  return pl.pallas_call(
      body, out_shape=jax.ShapeDtypeStruct((n, d), jnp.float32),
  )(x, p, w1, b1.reshape(1, -1), g.reshape(1, -1), be.reshape(1, -1),
    w2, b2.reshape(1, -1))


def _tc_layer_b(x, p, w1, b1, g, be, w2, b2, wl1, bl1, wl2, bl2):
  n, d = x.shape
  dout = wl2.shape[0]

  def body(x_ref, p_ref, w1_ref, b1_ref, g_ref, be_ref, w2_ref, b2_ref,
           wl1_ref, bl1_ref, wl2_ref, bl2_ref, o_ref):
    h = x_ref[...] + p_ref[0, :n, :] + p_ref[1, :n, :]
    h = _gin_mlp(h, w1_ref[...], b1_ref[...], g_ref[...], be_ref[...],
                 w2_ref[...], b2_ref[...])
    h = jnp.maximum(_matmul_t(h, wl1_ref[...]) + bl1_ref[...], 0.0)
    o_ref[...] = _matmul_t(h, wl2_ref[...]) + bl2_ref[...]

  return pl.pallas_call(
      body, out_shape=jax.ShapeDtypeStruct((n, dout), jnp.float32),
  )(x, p, w1, b1.reshape(1, -1), g.reshape(1, -1), be.reshape(1, -1),
    w2, b2.reshape(1, -1), wl1, bl1.reshape(1, -1), wl2, bl2.reshape(1, -1))


def kernel(x, edge_index, W1a, b1a, g1a, be1a, W2a, b2a,
           W1b, b1b, g1b, be1b, W2b, b2b, Wl1, bl1, Wl2, bl2):
  n = x.shape[0]
  ei_r, cpt8 = _pad_edges(edge_index, n)
  p = _sc_segment_sum(x, ei_r, cpt8)
  h = _tc_layer_a(x, p, W1a, b1a, g1a, be1a, W2a, b2a)
  q = _sc_segment_sum(h, ei_r, cpt8)
  return _tc_layer_b(h, q, W1b, b1b, g1b, be1b, W2b, b2b, Wl1, bl1, Wl2, bl2)


# speedup vs baseline: 1.1582x; 1.1582x over previous
import functools

import jax
import jax.numpy as jnp
from jax import lax
from jax.experimental import pallas as pl
from jax.experimental.pallas import tpu as pltpu
from jax.experimental.pallas import tpu_sc as plsc

_NC = 2
_NS = 16
_CH = 128
_RCH = 128


def _npad(n):
  return (n // (_NS * _RCH) + 1) * (_NS * _RCH)


def _pad_edges(edge_index, n):
  e = edge_index.shape[1]
  nw = _NC * _NS
  nchunks = -(-e // _CH)
  cpt8 = -(-(-(-nchunks // nw)) // 8) * 8
  epad = nw * cpt8 * _CH
  if epad > e:
    npad = _npad(n)
    idx = jnp.arange(epad - e, dtype=jnp.int32)
    dsts = n + idx % (npad - n)
    srcs = idx % n
    pad = jnp.stack([srcs, dsts], axis=0)
    edge_index = jnp.concatenate([edge_index, pad], axis=1)
  return edge_index, cpt8


def _sc_segment_sum(x, ei_r, cpt8):
  n, d = x.shape
  npad = _npad(n)
  rpt = npad // _NS
  nr = rpt // _RCH
  assert d % 16 == 0, d

  mesh = plsc.VectorSubcoreMesh(core_axis_name="c", subcore_axis_name="s")

  @functools.partial(
      pl.kernel,
      out_type=jax.ShapeDtypeStruct((_NC, npad, d), jnp.float32),
      mesh=mesh,
      scratch_types=[
          [pltpu.VMEM((_CH,), jnp.int32)] * 2,
          pltpu.VMEM((_CH,), jnp.int32),
          pltpu.VMEM((_CH, d), jnp.float32),
          pltpu.VMEM((_RCH, d), jnp.float32),
          pltpu.VMEM_SHARED((npad, d), jnp.float32),
          [pltpu.SemaphoreType.DMA] * 2,
          pltpu.SemaphoreType.DMA,
          pltpu.SemaphoreType.DMA,
      ],
  )
  def k(x_hbm, ei_hbm, out_hbm, sidx, didx, rows, zb, acc, semi, semd, semg):
    cid = lax.axis_index("c")
    sid = lax.axis_index("s")
    wid = sid * _NC + cid
    c0 = wid * cpt8

    nw = _NC * _NS

    def cb(t):
      return (wid + nw * t) * _CH

    def sidx_start(t, b):
      pltpu.async_copy(ei_hbm.at[0, pl.ds(cb(t), _CH)], sidx[b], semi[b])

    def sidx_wait(t, b):
      pltpu.make_async_copy(ei_hbm.at[0, pl.ds(cb(t), _CH)], sidx[b],
                            semi[b]).wait()

    def didx_start(t):
      pltpu.async_copy(ei_hbm.at[1, pl.ds(cb(t), _CH)], didx, semd)

    def didx_wait(t):
      pltpu.make_async_copy(ei_hbm.at[1, pl.ds(cb(t), _CH)], didx,
                            semd).wait()

    sidx_start(0, 0)
    sidx_start(1, 1)
    didx_start(0)


    def zrow(i, carry):
      for j in range(d // 16):
        zb[i, pl.ds(j * 16, 16)] = jnp.zeros((16,), jnp.float32)
      return carry
    lax.fori_loop(0, _RCH, zrow, 0)
    r0 = sid * rpt
    for kk in range(nr):
      pltpu.async_copy(zb, acc.at[pl.ds(r0 + kk * _RCH, _RCH)], semg)
    for kk in range(nr):
      pltpu.make_async_copy(zb, acc.at[pl.ds(r0 + kk * _RCH, _RCH)],
                            semg).wait()
    plsc.subcore_barrier()

    def body(g, carry):
      for b in range(2):
        t = 2 * g + b
        sidx_wait(t, b)
        didx_wait(t)
        pltpu.async_copy(x_hbm.at[sidx[b]], rows, semg).wait()
        pltpu.sync_copy(rows, acc.at[didx], add=True)

        @pl.when(t + 1 < cpt8)
        def _(t=t):
          didx_start(t + 1)

        @pl.when(t + 2 < cpt8)
        def _(t=t, b=b):
          sidx_start(t + 2, b)

      return carry
    lax.fori_loop(0, cpt8 // 2, body, 0)
    plsc.subcore_barrier()

    for kk in range(nr):
      pltpu.async_copy(acc.at[pl.ds(r0 + kk * _RCH, _RCH)],
                       out_hbm.at[cid, pl.ds(r0 + kk * _RCH, _RCH)], semg)
    for kk in range(nr):
      pltpu.make_async_copy(acc.at[pl.ds(r0 + kk * _RCH, _RCH)],
                            out_hbm.at[cid, pl.ds(r0 + kk * _RCH, _RCH)],
                            semg).wait()

  return k(x, ei_r)


def _matmul_t(h, w):
  return lax.dot_general(h, w, (((1,), (1,)), ((), ())),
                         preferred_element_type=jnp.float32)


def _gin_mlp(h, w1, b1, g, be, w2, b2):
  h = _matmul_t(h, w1) + b1
  m = jnp.mean(h, axis=0, keepdims=True)
  v = jnp.mean((h - m) * (h - m), axis=0, keepdims=True)
  h = (h - m) * lax.rsqrt(v + 1e-5) * g + be
  h = jnp.maximum(h, 0.0)
  h = _matmul_t(h, w2) + b2
  return jnp.maximum(h, 0.0)


def _tc_layer_a(x, p, w1, b1, g, be, w2, b2):
  n, d = x.shape

  def body(x_ref, p_ref, w1_ref, b1_ref, g_ref, be_ref, w2_ref, b2_ref, o_ref):
    h = x_ref[...] + p_ref[0, :n, :] + p_ref[1, :n, :]
    o_ref[...] = _gin_mlp(h, w1_ref[...], b1_ref[...], g_ref[...],
                          be_ref[...], w2_ref[...], b2_ref[...])

  return pl.pallas_call(
      body, out_shape=jax.ShapeDtypeStruct((n, d), jnp.float32),
  )(x, p, w1, b1.reshape(1, -1), g.reshape(1, -1), be.reshape(1, -1),
    w2, b2.reshape(1, -1))


def _tc_layer_b(x, p, w1, b1, g, be, w2, b2, wl1, bl1, wl2, bl2):
  n, d = x.shape
  dout = wl2.shape[0]

  def body(x_ref, p_ref, w1_ref, b1_ref, g_ref, be_ref, w2_ref, b2_ref,
           wl1_ref, bl1_ref, wl2_ref, bl2_ref, o_ref):
    h = x_ref[...] + p_ref[0, :n, :] + p_ref[1, :n, :]
    h = _gin_mlp(h, w1_ref[...], b1_ref[...], g_ref[...], be_ref[...],
                 w2_ref[...], b2_ref[...])
    h = jnp.maximum(_matmul_t(h, wl1_ref[...]) + bl1_ref[...], 0.0)
    o_ref[...] = _matmul_t(h, wl2_ref[...]) + bl2_ref[...]

  return pl.pallas_call(
      body, out_shape=jax.ShapeDtypeStruct((n, dout), jnp.float32),
  )(x, p, w1, b1.reshape(1, -1), g.reshape(1, -1), be.reshape(1, -1),
    w2, b2.reshape(1, -1), wl1, bl1.reshape(1, -1), wl2, bl2.reshape(1, -1))


def kernel(x, edge_index, W1a, b1a, g1a, be1a, W2a, b2a,
           W1b, b1b, g1b, be1b, W2b, b2b, Wl1, bl1, Wl2, bl2):
  n = x.shape[0]
  ei_r, cpt8 = _pad_edges(edge_index, n)
  p = _sc_segment_sum(x, ei_r, cpt8)
  h = _tc_layer_a(x, p, W1a, b1a, g1a, be1a, W2a, b2a)
  q = _sc_segment_sum(h, ei_r, cpt8)
  return _tc_layer_b(h, q, W1b, b1b, g1b, be1b, W2b, b2b, Wl1, bl1, Wl2, bl2)


# speedup vs baseline: 1.1597x; 1.0014x over previous
import functools

import jax
import jax.numpy as jnp
from jax import lax
from jax.experimental import pallas as pl
from jax.experimental.pallas import tpu as pltpu
from jax.experimental.pallas import tpu_sc as plsc

_NC = 2
_NS = 16
_CH = 128
_RCH = 128


def _npad(n):
  return (n // (_NS * _RCH) + 1) * (_NS * _RCH)


def _pad_edges(edge_index, n):
  e = edge_index.shape[1]
  nw = _NC * _NS
  nchunks = -(-e // _CH)
  cpt8 = -(-(-(-nchunks // nw)) // 8) * 8
  epad = nw * cpt8 * _CH
  if epad > e:
    npad = _npad(n)
    idx = jnp.arange(epad - e, dtype=jnp.int32)
    dsts = n + idx % (npad - n)
    srcs = idx % n
    pad = jnp.stack([srcs, dsts], axis=0)
    edge_index = jnp.concatenate([edge_index, pad], axis=1)
  return edge_index, cpt8


def _sc_segment_sum(x, ei_r, cpt8):
  n, d = x.shape
  npad = _npad(n)
  rpt = npad // _NS
  nr = rpt // _RCH
  assert d % 16 == 0, d

  mesh = plsc.VectorSubcoreMesh(core_axis_name="c", subcore_axis_name="s")

  @functools.partial(
      pl.kernel,
      out_type=jax.ShapeDtypeStruct((_NC, npad, d), jnp.float32),
      mesh=mesh,
      scratch_types=[
          [pltpu.VMEM((_CH,), jnp.int32)] * 2,
          pltpu.VMEM((_CH,), jnp.int32),
          pltpu.VMEM((_CH, d), jnp.float32),
          pltpu.VMEM((_RCH, d), jnp.float32),
          pltpu.VMEM_SHARED((npad, d), jnp.float32),
          [pltpu.SemaphoreType.DMA] * 2,
          pltpu.SemaphoreType.DMA,
          pltpu.SemaphoreType.DMA,
      ],
  )
  def k(x_hbm, ei_hbm, out_hbm, sidx, didx, rows, zb, acc, semi, semd, semg):
    cid = lax.axis_index("c")
    sid = lax.axis_index("s")
    wid = sid * _NC + cid
    c0 = wid * cpt8

    nw = _NC * _NS

    def cb(t):
      return (wid + nw * t) * _CH

    def sidx_start(t, b):
      pltpu.async_copy(ei_hbm.at[0, pl.ds(cb(t), _CH)], sidx[b], semi[b])

    def sidx_wait(t, b):
      pltpu.make_async_copy(ei_hbm.at[0, pl.ds(cb(t), _CH)], sidx[b],
                            semi[b]).wait()

    def didx_start(t):
      pltpu.async_copy(ei_hbm.at[1, pl.ds(cb(t), _CH)], didx, semd)

    def didx_wait(t):
      pltpu.make_async_copy(ei_hbm.at[1, pl.ds(cb(t), _CH)], didx,
                            semd).wait()

    sidx_start(0, 0)
    sidx_start(1, 1)
    didx_start(0)


    def zrow(i, carry):
      for j in range(d // 16):
        zb[i, pl.ds(j * 16, 16)] = jnp.zeros((16,), jnp.float32)
      return carry
    lax.fori_loop(0, _RCH, zrow, 0)
    r0 = sid * rpt
    for kk in range(nr):
      pltpu.sync_copy(zb, acc.at[pl.ds(r0 + kk * _RCH, _RCH)])
    plsc.subcore_barrier()

    def body(g, carry):
      for b in range(2):
        t = 2 * g + b
        sidx_wait(t, b)
        didx_wait(t)
        pltpu.async_copy(x_hbm.at[sidx[b]], rows, semg).wait()
        pltpu.sync_copy(rows, acc.at[didx], add=True)

        @pl.when(t + 1 < cpt8)
        def _(t=t):
          didx_start(t + 1)

        @pl.when(t + 2 < cpt8)
        def _(t=t, b=b):
          sidx_start(t + 2, b)

      return carry
    lax.fori_loop(0, cpt8 // 2, body, 0)
    plsc.subcore_barrier()

    for kk in range(nr):
      pltpu.sync_copy(acc.at[pl.ds(r0 + kk * _RCH, _RCH)],
                      out_hbm.at[cid, pl.ds(r0 + kk * _RCH, _RCH)])

  return k(x, ei_r)


def _matmul_t(h, w):
  return lax.dot_general(h, w, (((1,), (1,)), ((), ())),
                         preferred_element_type=jnp.float32)


def _gin_mlp(h, w1, b1, g, be, w2, b2):
  h = _matmul_t(h, w1) + b1
  m = jnp.mean(h, axis=0, keepdims=True)
  v = jnp.mean((h - m) * (h - m), axis=0, keepdims=True)
  h = (h - m) * lax.rsqrt(v + 1e-5) * g + be
  h = jnp.maximum(h, 0.0)
  h = _matmul_t(h, w2) + b2
  return jnp.maximum(h, 0.0)


def _tc_layer_a(x, p, w1, b1, g, be, w2, b2):
  n, d = x.shape

  def body(x_ref, p_ref, w1_ref, b1_ref, g_ref, be_ref, w2_ref, b2_ref, o_ref):
    h = x_ref[...] + p_ref[0, :n, :] + p_ref[1, :n, :]
    o_ref[...] = _gin_mlp(h, w1_ref[...], b1_ref[...], g_ref[...],
                          be_ref[...], w2_ref[...], b2_ref[...])

  return pl.pallas_call(
      body, out_shape=jax.ShapeDtypeStruct((n, d), jnp.float32),
  )(x, p, w1, b1.reshape(1, -1), g.reshape(1, -1), be.reshape(1, -1),
    w2, b2.reshape(1, -1))


def _tc_layer_b(x, p, w1, b1, g, be, w2, b2, wl1, bl1, wl2, bl2):
  n, d = x.shape
  dout = wl2.shape[0]

  def body(x_ref, p_ref, w1_ref, b1_ref, g_ref, be_ref, w2_ref, b2_ref,
           wl1_ref, bl1_ref, wl2_ref, bl2_ref, o_ref):
    h = x_ref[...] + p_ref[0, :n, :] + p_ref[1, :n, :]
    h = _gin_mlp(h, w1_ref[...], b1_ref[...], g_ref[...], be_ref[...],
                 w2_ref[...], b2_ref[...])
    h = jnp.maximum(_matmul_t(h, wl1_ref[...]) + bl1_ref[...], 0.0)
    o_ref[...] = _matmul_t(h, wl2_ref[...]) + bl2_ref[...]

  return pl.pallas_call(
      body, out_shape=jax.ShapeDtypeStruct((n, dout), jnp.float32),
  )(x, p, w1, b1.reshape(1, -1), g.reshape(1, -1), be.reshape(1, -1),
    w2, b2.reshape(1, -1), wl1, bl1.reshape(1, -1), wl2, bl2.reshape(1, -1))


def kernel(x, edge_index, W1a, b1a, g1a, be1a, W2a, b2a,
           W1b, b1b, g1b, be1b, W2b, b2b, Wl1, bl1, Wl2, bl2):
  n = x.shape[0]
  ei_r, cpt8 = _pad_edges(edge_index, n)
  p = _sc_segment_sum(x, ei_r, cpt8)
  h = _tc_layer_a(x, p, W1a, b1a, g1a, be1a, W2a, b2a)
  q = _sc_segment_sum(h, ei_r, cpt8)
  return _tc_layer_b(h, q, W1b, b1b, g1b, be1b, W2b, b2b, Wl1, bl1, Wl2, bl2)
